# baseline (speedup 1.0000x reference)
import jax
import jax.numpy as jnp
from jax import lax
from jax.experimental import pallas as pl
from jax.experimental.pallas import tpu as pltpu
from jax.experimental.pallas import tpu_sc as plsc

N = 16384
NNZ = 262144
NC = 2
NT = 16
SL = N // NT
CH = NNZ // (NC * NT)
NCH = 4
CW = CH // NCH
L = 16


def _body(pred_h, vidx_h, cidx_h, coeff_h, lb_h, ub_h,
          out_h,
          sl_a, sl_b, sl_c, vals_sl,
          values_v, vi_v, ci0_v, ci1_v, ci2_v, ci3_v, co_v,
          g0_v, g1_v, g2_v, g3_v,
          sem_vi, sem_co, sem_ci, sem_p, sem_lb, sem_ub, sem_sc,
          values_sh, ax_sh):
    ci_refs = (ci0_v, ci1_v, ci2_v, ci3_v)
    g_refs = (g0_v, g1_v, g2_v, g3_v)
    cid = lax.axis_index("c")
    tid = lax.axis_index("s")
    base = tid * SL
    cbase = (cid * NT + tid) * CH

    cp_vi = pltpu.async_copy(vidx_h.at[pl.ds(cbase, CH)], vi_v, sem_vi)
    cp_co = pltpu.async_copy(coeff_h.at[pl.ds(cbase, CH)], co_v, sem_co)
    cp_ci = [pltpu.async_copy(cidx_h.at[pl.ds(cbase + k * CW, CW)],
                              ci_refs[k], sem_ci) for k in range(NCH)]
    cp_p = pltpu.async_copy(pred_h.at[pl.ds(base, SL)], sl_a, sem_p)
    cp_lb = pltpu.async_copy(lb_h.at[pl.ds(base, SL)], sl_b, sem_lb)
    cp_ub = pltpu.async_copy(ub_h.at[pl.ds(base, SL)], sl_c, sem_ub)

    cp_p.wait()
    cp_lb.wait()
    cp_ub.wait()

    @plsc.parallel_loop(0, SL // L, 1, unroll=4)
    def p1(i):
        s = pl.ds(i * L, L)
        p, lo, hi = sl_a[s], sl_b[s], sl_c[s]
        vals_sl[s] = lo + p * (hi - lo)
        sl_a[s] = jnp.zeros((L,), jnp.float32)
    pltpu.sync_copy(vals_sl, values_sh.at[pl.ds(base, SL)])
    pltpu.sync_copy(sl_a, ax_sh.at[pl.ds(base, SL)])
    plsc.subcore_barrier()

    pltpu.sync_copy(values_sh, values_v)
    cp_vi.wait()
    cp_co.wait()
    for c in cp_ci:
        c.wait()

    scatters = []
    for k in range(NCH):
        gk = g_refs[k]

        @plsc.parallel_loop(0, CW // L, 1, unroll=4)
        def p2(i, k=k, gk=gk):
            s = pl.ds(i * L, L)
            f = pl.ds(k * CW + i * L, L)
            gk[s] = plsc.load_gather(values_v, [vi_v[f]]) * co_v[f]
        scatters.append(pltpu.async_copy(
            gk, ax_sh.at[ci_refs[k]], sem_sc, add=True))
    for d in scatters:
        d.wait()
    plsc.subcore_barrier()

    pltpu.sync_copy(ax_sh.at[pl.ds(base, SL)], out_h.at[cid, pl.ds(base, SL)])


_mesh = plsc.VectorSubcoreMesh(core_axis_name="c", subcore_axis_name="s")

_sc_call = pl.kernel(
    _body,
    out_type=jax.ShapeDtypeStruct((NC, N), jnp.float32),
    mesh=_mesh,
    compiler_params=pltpu.CompilerParams(needs_layout_passes=False),
    scratch_types=[
        pltpu.VMEM((SL,), jnp.float32),
        pltpu.VMEM((SL,), jnp.float32),
        pltpu.VMEM((SL,), jnp.float32),
        pltpu.VMEM((SL,), jnp.float32),
        pltpu.VMEM((N,), jnp.float32),
        pltpu.VMEM((CH,), jnp.int32),
        pltpu.VMEM((CW,), jnp.int32),
        pltpu.VMEM((CW,), jnp.int32),
        pltpu.VMEM((CW,), jnp.int32),
        pltpu.VMEM((CW,), jnp.int32),
        pltpu.VMEM((CH,), jnp.float32),
        pltpu.VMEM((CW,), jnp.float32),
        pltpu.VMEM((CW,), jnp.float32),
        pltpu.VMEM((CW,), jnp.float32),
        pltpu.VMEM((CW,), jnp.float32),
        pltpu.SemaphoreType.DMA,
        pltpu.SemaphoreType.DMA,
        pltpu.SemaphoreType.DMA,
        pltpu.SemaphoreType.DMA,
        pltpu.SemaphoreType.DMA,
        pltpu.SemaphoreType.DMA,
        pltpu.SemaphoreType.DMA,
        pltpu.VMEM_SHARED((N,), jnp.float32),
        pltpu.VMEM_SHARED((N,), jnp.float32),
    ],
)


def _fin_body(part_ref, rhs_ref, sen_ref, out_ref):
    ax = part_ref[0, :] + part_ref[1, :]
    d = ax - rhs_ref[...]
    sen = sen_ref[...]
    v = jnp.where(sen == 1, jnp.maximum(d, 0.0),
        jnp.where(sen == 2, jnp.maximum(-d, 0.0),
        jnp.where(sen == 3, jnp.abs(d),
                  jnp.zeros_like(d))))
    out_ref[...] = jnp.sum(v).reshape(1, 1)


_fin_call = pl.pallas_call(
    _fin_body,
    out_shape=jax.ShapeDtypeStruct((1, 1), jnp.float32),
)


def kernel(pred, constr_idx, var_idx, coeff, constr_rhs, constr_sense,
           n_vars, n_constrs, var_lb, var_ub):
    part = _sc_call(pred, var_idx.astype(jnp.int32),
                    constr_idx.astype(jnp.int32), coeff, var_lb, var_ub)
    tot = _fin_call(part, constr_rhs, constr_sense.astype(jnp.int32))
    return tot[0, 0] / n_constrs


# speedup vs baseline: 1.0035x; 1.0035x over previous
import jax
import jax.numpy as jnp
from jax import lax
from jax.experimental import pallas as pl
from jax.experimental.pallas import tpu as pltpu
from jax.experimental.pallas import tpu_sc as plsc

N = 16384
NNZ = 262144
NC = 2
NT = 16
SL = N // NT
CH = NNZ // (NC * NT)
NCH = 2
CW = CH // NCH
L = 16


def _body(pred_h, vidx_h, cidx_h, coeff_h, lb_h, ub_h,
          out_h,
          sl_a, sl_b, sl_c, vals_sl,
          values_v, vi_v, ci0_v, ci1_v, co_v, g0_v, g1_v,
          sem_vi, sem_co, sem_ci, sem_p, sem_lb, sem_ub, sem_sc,
          values_sh, ax_sh):
    ci_refs = (ci0_v, ci1_v)
    g_refs = (g0_v, g1_v)
    cid = lax.axis_index("c")
    tid = lax.axis_index("s")
    base = tid * SL
    cbase = (cid * NT + tid) * CH

    cp_vi = pltpu.async_copy(vidx_h.at[pl.ds(cbase, CH)], vi_v, sem_vi)
    cp_co = pltpu.async_copy(coeff_h.at[pl.ds(cbase, CH)], co_v, sem_co)
    cp_ci = [pltpu.async_copy(cidx_h.at[pl.ds(cbase + k * CW, CW)],
                              ci_refs[k], sem_ci) for k in range(NCH)]
    cp_p = pltpu.async_copy(pred_h.at[pl.ds(base, SL)], sl_a, sem_p)
    cp_lb = pltpu.async_copy(lb_h.at[pl.ds(base, SL)], sl_b, sem_lb)
    cp_ub = pltpu.async_copy(ub_h.at[pl.ds(base, SL)], sl_c, sem_ub)

    cp_p.wait()
    cp_lb.wait()
    cp_ub.wait()

    @plsc.parallel_loop(0, SL // L, 1, unroll=4)
    def p1(i):
        s = pl.ds(i * L, L)
        p, lo, hi = sl_a[s], sl_b[s], sl_c[s]
        vals_sl[s] = lo + p * (hi - lo)
        sl_a[s] = jnp.zeros((L,), jnp.float32)
    pltpu.sync_copy(vals_sl, values_sh.at[pl.ds(base, SL)])
    pltpu.sync_copy(sl_a, ax_sh.at[pl.ds(base, SL)])
    plsc.subcore_barrier()

    pltpu.sync_copy(values_sh, values_v)
    cp_vi.wait()
    cp_co.wait()
    for c in cp_ci:
        c.wait()

    scatters = []
    for k in range(NCH):
        gk = g_refs[k]

        @plsc.parallel_loop(0, CW // L, 1, unroll=4)
        def p2(i, k=k, gk=gk):
            s = pl.ds(i * L, L)
            f = pl.ds(k * CW + i * L, L)
            gk[s] = plsc.load_gather(values_v, [vi_v[f]]) * co_v[f]
        scatters.append(pltpu.async_copy(
            gk, ax_sh.at[ci_refs[k]], sem_sc, add=True))
    for d in scatters:
        d.wait()
    plsc.subcore_barrier()

    pltpu.sync_copy(ax_sh.at[pl.ds(base, SL)], out_h.at[cid, pl.ds(base, SL)])


_mesh = plsc.VectorSubcoreMesh(core_axis_name="c", subcore_axis_name="s")

_sc_call = pl.kernel(
    _body,
    out_type=jax.ShapeDtypeStruct((NC, N), jnp.float32),
    mesh=_mesh,
    compiler_params=pltpu.CompilerParams(needs_layout_passes=False),
    scratch_types=[
        pltpu.VMEM((SL,), jnp.float32),
        pltpu.VMEM((SL,), jnp.float32),
        pltpu.VMEM((SL,), jnp.float32),
        pltpu.VMEM((SL,), jnp.float32),
        pltpu.VMEM((N,), jnp.float32),
        pltpu.VMEM((CH,), jnp.int32),
        pltpu.VMEM((CW,), jnp.int32),
        pltpu.VMEM((CW,), jnp.int32),
        pltpu.VMEM((CH,), jnp.float32),
        pltpu.VMEM((CW,), jnp.float32),
        pltpu.VMEM((CW,), jnp.float32),
        pltpu.SemaphoreType.DMA,
        pltpu.SemaphoreType.DMA,
        pltpu.SemaphoreType.DMA,
        pltpu.SemaphoreType.DMA,
        pltpu.SemaphoreType.DMA,
        pltpu.SemaphoreType.DMA,
        pltpu.SemaphoreType.DMA,
        pltpu.VMEM_SHARED((N,), jnp.float32),
        pltpu.VMEM_SHARED((N,), jnp.float32),
    ],
)


def _fin_body(part_ref, rhs_ref, sen_ref, out_ref):
    ax = part_ref[0, :] + part_ref[1, :]
    d = ax - rhs_ref[...]
    sen = sen_ref[...]
    v = jnp.where(sen == 1, jnp.maximum(d, 0.0),
        jnp.where(sen == 2, jnp.maximum(-d, 0.0),
        jnp.where(sen == 3, jnp.abs(d),
                  jnp.zeros_like(d))))
    out_ref[...] = jnp.sum(v).reshape(1, 1)


_fin_call = pl.pallas_call(
    _fin_body,
    out_shape=jax.ShapeDtypeStruct((1, 1), jnp.float32),
)


def kernel(pred, constr_idx, var_idx, coeff, constr_rhs, constr_sense,
           n_vars, n_constrs, var_lb, var_ub):
    part = _sc_call(pred, var_idx.astype(jnp.int32),
                    constr_idx.astype(jnp.int32), coeff, var_lb, var_ub)
    tot = _fin_call(part, constr_rhs, constr_sense.astype(jnp.int32))
    return tot[0, 0] / n_constrs
